# baseline (speedup 1.0000x reference)
import jax
import jax.numpy as jnp
from jax.experimental import pallas as pl

_K = 32
_BLOCK = 1000


def _mlp_kernel(a_ref, w1_ref, b1_ref, we1_ref, be1_ref, we2_ref, be2_ref,
                wd1_ref, bd1_ref, wd2_ref, bd2_ref, wo_ref, bo_ref, out_ref):
    f32 = jnp.float32
    a = a_ref[...]
    h = jnp.tanh(jnp.dot(a, w1_ref[...], preferred_element_type=f32) + b1_ref[...])
    h = jnp.tanh(jnp.dot(h, we1_ref[...], preferred_element_type=f32) + be1_ref[...])
    e = jnp.tanh(jnp.dot(h, we2_ref[...], preferred_element_type=f32) + be2_ref[...])
    h = jnp.tanh(jnp.dot(e, wd1_ref[...], preferred_element_type=f32) + bd1_ref[...])
    d = jnp.dot(h, wd2_ref[...], preferred_element_type=f32) + bd2_ref[...]
    t = jnp.tanh(d)
    o = jnp.sum(t * wo_ref[...], axis=1, keepdims=True) + bo_ref[...]
    out_ref[...] = jax.nn.sigmoid(o)


def kernel(x, edge_index, edge_attr, W_ih, W_hh, b_ih, b_hh, W1, b1, Wg, bg,
           We1, be1, We2, be2, Wd1, bd1, Wd2, bd2, Wo, bo):
    a = edge_attr.reshape(-1, _K)
    rows = a.shape[0]

    w1t = W1.T
    we1t = We1.T
    we2t = jnp.pad(We2.T, ((0, 0), (0, 2)))
    be2p = jnp.pad(be2, (0, 2))[None]
    wd1t = jnp.pad(Wd1.T, ((0, 2), (0, 0)))
    wd2t = Wd2.T

    def full(shape):
        return pl.BlockSpec(shape, lambda i: (0,) * len(shape))

    out = pl.pallas_call(
        _mlp_kernel,
        grid=(rows // _BLOCK,),
        in_specs=[
            pl.BlockSpec((_BLOCK, _K), lambda i: (i, 0)),
            full((_K, 256)), full((1, 256)),
            full((256, 128)), full((1, 128)),
            full((128, 8)), full((1, 8)),
            full((8, 128)), full((1, 128)),
            full((128, 256)), full((1, 256)),
            full((1, 256)), full((1, 1)),
        ],
        out_specs=pl.BlockSpec((_BLOCK, 1), lambda i: (i, 0)),
        out_shape=jax.ShapeDtypeStruct((rows, 1), jnp.float32),
    )(a, w1t, b1[None], we1t, be1[None], we2t, be2p,
      wd1t, bd1[None], wd2t, bd2[None], Wo, bo[None])
    return out[:, 0]


# speedup vs baseline: 1.1090x; 1.1090x over previous
import jax
import jax.numpy as jnp
from jax.experimental import pallas as pl
from jax.experimental.pallas import tpu as pltpu

_K = 32
_BLOCK = 1000

_DN = (((1,), (1,)), ((), ()))


def _mlp_kernel(a_ref, w1_ref, b1_ref, we1_ref, be1_ref, we2_ref, be2_ref,
                wd1_ref, bd1_ref, wd2_ref, bd2_ref, wo_ref, bo_ref, out_ref):
    f32 = jnp.float32

    def lin(v, w_ref, b_ref):
        return jax.lax.dot_general(v, w_ref[...], _DN,
                                   preferred_element_type=f32) + b_ref[...]

    a = a_ref[...]
    h = jnp.tanh(lin(a, w1_ref, b1_ref))
    h = jnp.tanh(lin(h, we1_ref, be1_ref))
    e = jnp.tanh(lin(h, we2_ref, be2_ref))
    h = jnp.tanh(lin(e, wd1_ref, bd1_ref))
    d = lin(h, wd2_ref, bd2_ref)
    t = jnp.tanh(d)
    o = jnp.sum(t * wo_ref[...], axis=1, keepdims=True) + bo_ref[...]
    out_ref[...] = jax.nn.sigmoid(o)


def kernel(x, edge_index, edge_attr, W_ih, W_hh, b_ih, b_hh, W1, b1, Wg, bg,
           We1, be1, We2, be2, Wd1, bd1, Wd2, bd2, Wo, bo):
    a = edge_attr.reshape(-1, _K)
    rows = a.shape[0]

    def full(shape):
        return pl.BlockSpec(shape, lambda i: (0,) * len(shape))

    out = pl.pallas_call(
        _mlp_kernel,
        grid=(rows // _BLOCK,),
        in_specs=[
            pl.BlockSpec((_BLOCK, _K), lambda i: (i, 0)),
            full((256, _K)), full((1, 256)),
            full((128, 256)), full((1, 128)),
            full((6, 128)), full((1, 6)),
            full((128, 6)), full((1, 128)),
            full((256, 128)), full((1, 256)),
            full((1, 256)), full((1, 1)),
        ],
        out_specs=pl.BlockSpec((_BLOCK, 1), lambda i: (i, 0)),
        out_shape=jax.ShapeDtypeStruct((rows, 1), jnp.float32),
        compiler_params=pltpu.CompilerParams(
            dimension_semantics=("parallel",)),
    )(a, W1, b1[None], We1, be1[None], We2, be2[None],
      Wd1, bd1[None], Wd2, bd2[None], Wo, bo[None])
    return out[:, 0]


# speedup vs baseline: 1.1800x; 1.0640x over previous
import jax
import jax.numpy as jnp
from jax.experimental import pallas as pl
from jax.experimental.pallas import tpu as pltpu

_K = 32
_BLOCK = 2000

_DN = (((1,), (1,)), ((), ()))


def _mlp_kernel(a_ref, w1_ref, b1_ref, we1_ref, be1_ref, we2_ref, be2_ref,
                wd1_ref, bd1_ref, wd2_ref, bd2_ref, wo_ref, bo_ref, out_ref):
    f32 = jnp.float32

    def lin(v, w_ref, b_ref):
        return jax.lax.dot_general(v, w_ref[...], _DN,
                                   preferred_element_type=f32) + b_ref[...]

    a = a_ref[...]
    h = jnp.tanh(lin(a, w1_ref, b1_ref))
    h = jnp.tanh(lin(h, we1_ref, be1_ref))
    e = jnp.tanh(lin(h, we2_ref, be2_ref))
    h = jnp.tanh(lin(e, wd1_ref, bd1_ref))
    d = lin(h, wd2_ref, bd2_ref)
    t = jnp.tanh(d)
    o = jnp.sum(t * wo_ref[...], axis=1, keepdims=True) + bo_ref[...]
    out_ref[...] = jax.nn.sigmoid(o)


def kernel(x, edge_index, edge_attr, W_ih, W_hh, b_ih, b_hh, W1, b1, Wg, bg,
           We1, be1, We2, be2, Wd1, bd1, Wd2, bd2, Wo, bo):
    a = edge_attr.reshape(-1, _K)
    rows = a.shape[0]

    def full(shape):
        return pl.BlockSpec(shape, lambda i: (0,) * len(shape))

    out = pl.pallas_call(
        _mlp_kernel,
        grid=(rows // _BLOCK,),
        in_specs=[
            pl.BlockSpec((_BLOCK, _K), lambda i: (i, 0)),
            full((256, _K)), full((1, 256)),
            full((128, 256)), full((1, 128)),
            full((6, 128)), full((1, 6)),
            full((128, 6)), full((1, 128)),
            full((256, 128)), full((1, 256)),
            full((1, 256)), full((1, 1)),
        ],
        out_specs=pl.BlockSpec((_BLOCK, 1), lambda i: (i, 0)),
        out_shape=jax.ShapeDtypeStruct((rows, 1), jnp.float32),
        compiler_params=pltpu.CompilerParams(
            dimension_semantics=("parallel",)),
    )(a, W1, b1[None], We1, be1[None], We2, be2[None],
      Wd1, bd1[None], Wd2, bd2[None], Wo, bo[None])
    return out[:, 0]
